# Initial kernel scaffold; baseline (speedup 1.0000x reference)
import jax
import jax.numpy as jnp
from jax.experimental import pallas as pl


def kernel(x, conv1_w, conv1_b, bn1_g, bn1_b, b0_l0_bn_g, b0_l0_bn_b, b0_l0_w, b0_l1_bn_g, b0_l1_bn_b, b0_l1_w, b0_l2_bn_g, b0_l2_bn_b, b0_l2_w, b0_l3_bn_g, b0_l3_bn_b, b0_l3_w, t0_bn_g, t0_bn_b, t0_w, t0_b, b1_l0_bn_g, b1_l0_bn_b, b1_l0_w, b1_l1_bn_g, b1_l1_bn_b, b1_l1_w, b1_l2_bn_g, b1_l2_bn_b, b1_l2_w, b1_l3_bn_g, b1_l3_bn_b, b1_l3_w, t1_bn_g, t1_bn_b, t1_w, t1_b, b2_l0_bn_g, b2_l0_bn_b, b2_l0_w, b2_l1_bn_g, b2_l1_bn_b, b2_l1_w, b2_l2_bn_g, b2_l2_bn_b, b2_l2_w, b2_l3_bn_g, b2_l3_bn_b, b2_l3_w, t2_bn_g, t2_bn_b, t2_w, t2_b, b3_l0_bn_g, b3_l0_bn_b, b3_l0_w, b3_l1_bn_g, b3_l1_bn_b, b3_l1_w, b3_l2_bn_g, b3_l2_bn_b, b3_l2_w, b3_l3_bn_g, b3_l3_bn_b, b3_l3_w, bn2_g, bn2_b, fc_w, fc_b):
    raise NotImplementedError("write your pallas kernel here")


# speedup vs baseline: 1.6656x; 1.6656x over previous
import functools

import jax
import jax.numpy as jnp
from jax import lax
from jax.experimental import pallas as pl
from jax.experimental.pallas import tpu as pltpu


def _ru(x, m):
    return (x + m - 1) // m * m


def _scale_bias(csum, csumsq, count, gamma, beta, eps=1e-5):
    mean = csum / count
    var = csumsq / count - mean * mean
    s = gamma * lax.rsqrt(var + eps)
    return s, beta - mean * s


def _stats2(y):
    return jnp.concatenate([jnp.sum(y, axis=0, keepdims=True),
                            jnp.sum(y * y, axis=0, keepdims=True)], axis=0)


def _stem_mm_kernel(a_ref, b_ref, bias_ref, o_ref, st_ref):
    y = jnp.dot(a_ref[...], b_ref[...],
                preferred_element_type=jnp.float32) + bias_ref[...]
    o_ref[...] = y
    st_ref[0] = _stats2(y)


def _stem_conv(patches, wmat, bias):
    M, K = patches.shape
    N = wmat.shape[1]
    tm = 512
    grid = M // tm
    out, st = pl.pallas_call(
        _stem_mm_kernel,
        out_shape=(jax.ShapeDtypeStruct((M, N), jnp.float32),
                   jax.ShapeDtypeStruct((grid, 2, N), jnp.float32)),
        grid=(grid,),
        in_specs=[pl.BlockSpec((tm, K), lambda i: (i, 0)),
                  pl.BlockSpec((K, N), lambda i: (0, 0)),
                  pl.BlockSpec((1, N), lambda i: (0, 0))],
        out_specs=(pl.BlockSpec((tm, N), lambda i: (i, 0)),
                   pl.BlockSpec((1, 2, N), lambda i: (i, 0, 0))),
        compiler_params=pltpu.CompilerParams(
            dimension_semantics=("parallel",)),
    )(patches, wmat, bias.astype(jnp.float32).reshape(1, N))
    return out, jnp.sum(st, axis=0)


def _stem_pool_kernel(x_ref, s_ref, b_ref, o_ref, st_ref, *, B):
    C = x_ref.shape[-1]
    a = jnp.maximum(x_ref[...] * s_ref[...] + b_ref[...], 0.0)
    v = a.reshape(B * 24, 2, 48, C)
    pm = jnp.max(v, axis=1)
    od = v[:, 1].reshape(B, 24, 48, C)
    sh = jnp.concatenate([jnp.zeros_like(od[:, :1]), od[:, :-1]],
                         axis=1).reshape(B * 24, 48, C)
    rr = jnp.maximum(pm, sh)
    vw = rr.reshape(B * 24, 24, 2, C)
    pmw = jnp.max(vw, axis=2)
    odw = vw[:, :, 1, :]
    shw = jnp.concatenate([jnp.zeros_like(odw[:, :1]), odw[:, :-1]], axis=1)
    y16 = jnp.maximum(pmw, shw).reshape(B * 576, C).astype(jnp.bfloat16)
    o_ref[...] = y16
    st_ref[0] = _stats2(y16.astype(jnp.float32))


def _stem_pool(conv_out, scale, bias, n, B):
    C = conv_out.shape[-1]
    grid = n // B
    out, st = pl.pallas_call(
        functools.partial(_stem_pool_kernel, B=B),
        out_shape=(jax.ShapeDtypeStruct((n * 576, C), jnp.bfloat16),
                   jax.ShapeDtypeStruct((grid, 2, C), jnp.float32)),
        grid=(grid,),
        in_specs=[pl.BlockSpec((B * 2304, C), lambda i: (i, 0)),
                  pl.BlockSpec((1, C), lambda i: (0, 0)),
                  pl.BlockSpec((1, C), lambda i: (0, 0))],
        out_specs=(pl.BlockSpec((B * 576, C), lambda i: (i, 0)),
                   pl.BlockSpec((1, 2, C), lambda i: (i, 0, 0))),
        compiler_params=pltpu.CompilerParams(
            dimension_semantics=("parallel",)),
    )(conv_out, scale.astype(jnp.float32).reshape(1, C),
      bias.astype(jnp.float32).reshape(1, C))
    return out, jnp.sum(st, axis=0)


def _dense_kernel(x_ref, w_ref, s_ref, b_ref, m_ref, o_ref, st_ref, pad_ref,
                  *, width, g):
    rows = x_ref.shape[0]
    off = pad_ref.shape[0] - rows - (width + 1)

    @pl.when(pl.program_id(0) == 0)
    def _():
        pad_ref[...] = jnp.zeros_like(pad_ref)

    x = x_ref[...]
    a = jnp.maximum(x.astype(jnp.float32) * s_ref[...] + b_ref[...], 0.0)
    pad_ref[off:off + rows, :] = a.astype(jnp.bfloat16)

    acc = jnp.zeros((rows, g), jnp.float32)
    for di in range(3):
        for dj in range(3):
            tap = di * 3 + dj
            shift = off + (di - 1) * width + (dj - 1)
            src = pad_ref[shift:shift + rows, :]
            part = jnp.dot(src, w_ref[tap], preferred_element_type=jnp.float32)
            ok = m_ref[:, tap:tap + 1] > 0.5
            acc = acc + jnp.where(ok, part, 0.0)

    y16 = acc.astype(jnp.bfloat16)
    o_ref[...] = jnp.concatenate([y16, x], axis=-1)
    st_ref[0] = _stats2(y16.astype(jnp.float32))


def _tap_masks(h, w, B):
    r = jnp.repeat(jnp.arange(h), w)
    c = jnp.tile(jnp.arange(w), h)
    cols = []
    for di in (-1, 0, 1):
        for dj in (-1, 0, 1):
            cols.append((r + di >= 0) & (r + di < h) &
                        (c + dj >= 0) & (c + dj < w))
    m = jnp.stack(cols, axis=1).astype(jnp.float32)
    return jnp.tile(m, (B, 1))


def _dense_layer(slab, w_oihw, scale, bias, h, w, B, n):
    rows_total, C = slab.shape
    g = w_oihw.shape[0]
    m = h * w
    R = B * m
    off = _ru(w + 1, 8)
    pad_rows = off + R + w + 1
    grid = n // B
    wt = jnp.transpose(w_oihw, (2, 3, 1, 0)).reshape(9, C, g).astype(jnp.bfloat16)
    masks = _tap_masks(h, w, B)
    out, st = pl.pallas_call(
        functools.partial(_dense_kernel, width=w, g=g),
        out_shape=(jax.ShapeDtypeStruct((rows_total, C + g), jnp.bfloat16),
                   jax.ShapeDtypeStruct((grid, 2, g), jnp.float32)),
        grid_spec=pltpu.PrefetchScalarGridSpec(
            num_scalar_prefetch=0,
            grid=(grid,),
            in_specs=[pl.BlockSpec((R, C), lambda i: (i, 0)),
                      pl.BlockSpec((9, C, g), lambda i: (0, 0, 0)),
                      pl.BlockSpec((1, C), lambda i: (0, 0)),
                      pl.BlockSpec((1, C), lambda i: (0, 0)),
                      pl.BlockSpec((R, 9), lambda i: (0, 0))],
            out_specs=(pl.BlockSpec((R, C + g), lambda i: (i, 0)),
                       pl.BlockSpec((1, 2, g), lambda i: (i, 0, 0))),
            scratch_shapes=[pltpu.VMEM((pad_rows, C), jnp.bfloat16)],
        ),
        compiler_params=pltpu.CompilerParams(
            dimension_semantics=("parallel",)),
    )(slab, wt, scale.astype(jnp.float32).reshape(1, C),
      bias.astype(jnp.float32).reshape(1, C), masks)
    return out, jnp.sum(st, axis=0)


def _trans_kernel(x_ref, w_ref, s_ref, b_ref, ob_ref, o_ref, st_ref, *, B, h, w):
    a = jnp.maximum(x_ref[...].astype(jnp.float32) * s_ref[...] + b_ref[...],
                    0.0).astype(jnp.bfloat16)
    y = jnp.dot(a, w_ref[...], preferred_element_type=jnp.float32) + ob_ref[...]
    C = y.shape[-1]
    t = y.reshape(B * h // 2, 2, w, C).sum(axis=1)
    t = t.reshape(B * h // 2, w // 2, 2, C).sum(axis=2)
    y16 = (t.reshape(B * h * w // 4, C) * 0.25).astype(jnp.bfloat16)
    o_ref[...] = y16
    st_ref[0] = _stats2(y16.astype(jnp.float32))


def _transition(slab, w_oihw, cbias, scale, bias, h, w, B, n):
    rows_total, C = slab.shape
    Cout = w_oihw.shape[0]
    m = h * w
    grid = n // B
    wmat = w_oihw.reshape(Cout, C).T.astype(jnp.bfloat16)
    out, st = pl.pallas_call(
        functools.partial(_trans_kernel, B=B, h=h, w=w),
        out_shape=(jax.ShapeDtypeStruct((rows_total // 4, Cout), jnp.bfloat16),
                   jax.ShapeDtypeStruct((grid, 2, Cout), jnp.float32)),
        grid=(grid,),
        in_specs=[pl.BlockSpec((B * m, C), lambda i: (i, 0)),
                  pl.BlockSpec((C, Cout), lambda i: (0, 0)),
                  pl.BlockSpec((1, C), lambda i: (0, 0)),
                  pl.BlockSpec((1, C), lambda i: (0, 0)),
                  pl.BlockSpec((1, Cout), lambda i: (0, 0))],
        out_specs=(pl.BlockSpec((B * m // 4, Cout), lambda i: (i, 0)),
                   pl.BlockSpec((1, 2, Cout), lambda i: (i, 0, 0))),
        compiler_params=pltpu.CompilerParams(
            dimension_semantics=("parallel",)),
    )(slab, wmat, scale.astype(jnp.float32).reshape(1, C),
      bias.astype(jnp.float32).reshape(1, C),
      cbias.astype(jnp.float32).reshape(1, Cout))
    return out, jnp.sum(st, axis=0)


def _head_kernel(x_ref, s_ref, b_ref, w_ref, ob_ref, o_ref):
    rows, C = x_ref.shape
    a = jnp.maximum(x_ref[...].astype(jnp.float32) * s_ref[...] + b_ref[...], 0.0)
    f = a.reshape(rows // 9, 9, C).sum(axis=1) * (1.0 / 9.0)
    o_ref[...] = jnp.dot(f.astype(jnp.bfloat16), w_ref[...],
                         preferred_element_type=jnp.float32) + ob_ref[...]


def _head(slab, scale, bias, fc_w, fc_b, n):
    rows_total, C = slab.shape
    ncls = fc_w.shape[0]
    Np = _ru(ncls, 128)
    wmat = jnp.pad(fc_w.T.astype(jnp.bfloat16), ((0, 0), (0, Np - ncls)))
    ob = jnp.pad(fc_b.astype(jnp.float32), (0, Np - ncls)).reshape(1, Np)
    out = pl.pallas_call(
        _head_kernel,
        out_shape=jax.ShapeDtypeStruct((n, Np), jnp.float32),
        grid=(2,),
        in_specs=[pl.BlockSpec((rows_total // 2, C), lambda i: (i, 0)),
                  pl.BlockSpec((1, C), lambda i: (0, 0)),
                  pl.BlockSpec((1, C), lambda i: (0, 0)),
                  pl.BlockSpec((C, Np), lambda i: (0, 0)),
                  pl.BlockSpec((1, Np), lambda i: (0, 0))],
        out_specs=pl.BlockSpec((n // 2, Np), lambda i: (i, 0)),
        compiler_params=pltpu.CompilerParams(
            dimension_semantics=("parallel",)),
    )(slab, scale.astype(jnp.float32).reshape(1, C),
      bias.astype(jnp.float32).reshape(1, C), wmat, ob)
    return out[:, :ncls]


def _stem_patches(x_nhwc):
    n, h, w, c = x_nhwc.shape
    xb = x_nhwc.astype(jnp.bfloat16)
    xp = jnp.pad(xb, ((0, 0), (3, 3), (3, 3), (0, 0)))
    ho = wo = h // 2
    cols = []
    for i in range(7):
        for j in range(7):
            cols.append(lax.slice(xp, (0, i, j, 0),
                                  (n, i + 2 * ho - 1, j + 2 * wo - 1, c),
                                  (1, 2, 2, 1)))
    return jnp.concatenate(cols, axis=-1).reshape(n * ho * wo, 49 * c)


def kernel(x, conv1_w, conv1_b, bn1_g, bn1_b, b0_l0_bn_g, b0_l0_bn_b, b0_l0_w, b0_l1_bn_g, b0_l1_bn_b, b0_l1_w, b0_l2_bn_g, b0_l2_bn_b, b0_l2_w, b0_l3_bn_g, b0_l3_bn_b, b0_l3_w, t0_bn_g, t0_bn_b, t0_w, t0_b, b1_l0_bn_g, b1_l0_bn_b, b1_l0_w, b1_l1_bn_g, b1_l1_bn_b, b1_l1_w, b1_l2_bn_g, b1_l2_bn_b, b1_l2_w, b1_l3_bn_g, b1_l3_bn_b, b1_l3_w, t1_bn_g, t1_bn_b, t1_w, t1_b, b2_l0_bn_g, b2_l0_bn_b, b2_l0_w, b2_l1_bn_g, b2_l1_bn_b, b2_l1_w, b2_l2_bn_g, b2_l2_bn_b, b2_l2_w, b2_l3_bn_g, b2_l3_bn_b, b2_l3_w, t2_bn_g, t2_bn_b, t2_w, t2_b, b3_l0_bn_g, b3_l0_bn_b, b3_l0_w, b3_l1_bn_g, b3_l1_bn_b, b3_l1_w, b3_l2_bn_g, b3_l2_bn_b, b3_l2_w, b3_l3_bn_g, b3_l3_bn_b, b3_l3_w, bn2_g, bn2_b, fc_w, fc_b):
    n = x.shape[0]

    x_nhwc = jnp.transpose(x.astype(jnp.float32), (0, 2, 3, 1))
    patches = _stem_patches(x_nhwc)
    wmat = jnp.transpose(conv1_w, (2, 3, 1, 0)).reshape(-1, 64).astype(jnp.bfloat16)
    conv_out, st = _stem_conv(patches, wmat, conv1_b)
    s1, b1 = _scale_bias(st[0], st[1], float(n * 48 * 48), bn1_g, bn1_b)
    slab, st0 = _stem_pool(conv_out, s1, b1, n, B=8)

    seg_sums, seg_sqs = [st0[0]], [st0[1]]
    count = float(n * 576)
    h = w = 24

    blocks = [
        [(b0_l0_bn_g, b0_l0_bn_b, b0_l0_w), (b0_l1_bn_g, b0_l1_bn_b, b0_l1_w),
         (b0_l2_bn_g, b0_l2_bn_b, b0_l2_w), (b0_l3_bn_g, b0_l3_bn_b, b0_l3_w)],
        [(b1_l0_bn_g, b1_l0_bn_b, b1_l0_w), (b1_l1_bn_g, b1_l1_bn_b, b1_l1_w),
         (b1_l2_bn_g, b1_l2_bn_b, b1_l2_w), (b1_l3_bn_g, b1_l3_bn_b, b1_l3_w)],
        [(b2_l0_bn_g, b2_l0_bn_b, b2_l0_w), (b2_l1_bn_g, b2_l1_bn_b, b2_l1_w),
         (b2_l2_bn_g, b2_l2_bn_b, b2_l2_w), (b2_l3_bn_g, b2_l3_bn_b, b2_l3_w)],
        [(b3_l0_bn_g, b3_l0_bn_b, b3_l0_w), (b3_l1_bn_g, b3_l1_bn_b, b3_l1_w),
         (b3_l2_bn_g, b3_l2_bn_b, b3_l2_w), (b3_l3_bn_g, b3_l3_bn_b, b3_l3_w)],
    ]
    transitions = [(t0_bn_g, t0_bn_b, t0_w, t0_b),
                   (t1_bn_g, t1_bn_b, t1_w, t1_b),
                   (t2_bn_g, t2_bn_b, t2_w, t2_b)]
    dense_B = [8, 16, 64, 128]
    trans_B = [8, 16, 64]

    for bi, layers in enumerate(blocks):
        for (g_, b_, w_) in layers:
            gsum = jnp.concatenate(seg_sums)
            gsq = jnp.concatenate(seg_sqs)
            sc, bs = _scale_bias(gsum, gsq, count, g_, b_)
            slab, st = _dense_layer(slab, w_, sc, bs, h, w, dense_B[bi], n)
            seg_sums.insert(0, st[0])
            seg_sqs.insert(0, st[1])
        if bi < 3:
            tg, tb, tw, tbb = transitions[bi]
            gsum = jnp.concatenate(seg_sums)
            gsq = jnp.concatenate(seg_sqs)
            sc, bs = _scale_bias(gsum, gsq, count, tg, tb)
            slab, st = _transition(slab, tw, tbb, sc, bs, h, w, trans_B[bi], n)
            h //= 2
            w //= 2
            count = float(n * h * w)
            seg_sums, seg_sqs = [st[0]], [st[1]]

    gsum = jnp.concatenate(seg_sums)
    gsq = jnp.concatenate(seg_sqs)
    sc, bs = _scale_bias(gsum, gsq, count, bn2_g, bn2_b)
    return _head(slab, sc, bs, fc_w, fc_b, n)


# speedup vs baseline: 3.6276x; 2.1779x over previous
import functools

import jax
import jax.numpy as jnp
from jax import lax
from jax.experimental import pallas as pl
from jax.experimental.pallas import tpu as pltpu


def _ru(x, m):
    return (x + m - 1) // m * m


def _scale_bias(csum, csumsq, count, gamma, beta, eps=1e-5):
    mean = csum / count
    var = csumsq / count - mean * mean
    s = gamma * lax.rsqrt(var + eps)
    return s, beta - mean * s


def _stats2(y):
    return jnp.concatenate([jnp.sum(y, axis=0, keepdims=True),
                            jnp.sum(y * y, axis=0, keepdims=True)], axis=0)


def _stem_conv_kernel(x_ref, w_ref, bias_ref, o_ref, st_ref, pad_ref, *, B):
    R = x_ref.shape[0]
    off = pad_ref.shape[0] - R - 57
    pad_ref[off:off + R, :] = x_ref[...]
    acc = jnp.zeros((R, o_ref.shape[-1]), jnp.float32)
    for di in (-2, -1, 0, 1):
        for dj in (-2, -1, 0, 1):
            tap = (di + 2) * 4 + (dj + 2)
            s = off + di * 56 + dj
            acc = acc + jnp.dot(pad_ref[s:s + R, :], w_ref[tap],
                                preferred_element_type=jnp.float32)
    y = acc + bias_ref[...]
    yv = y.reshape(B, 51, 56, y.shape[-1])[:, 2:50, 2:50, :]
    yv = yv.reshape(B * 2304, y.shape[-1])
    o_ref[...] = yv
    st_ref[0] = _stats2(yv)


def _stem_w2(conv1_w):
    zero = jnp.zeros((64,), conv1_w.dtype)
    taps = []
    for di in (-2, -1, 0, 1):
        for dj in (-2, -1, 0, 1):
            rows = []
            for c in range(3):
                for pr in (0, 1):
                    for pc in (0, 1):
                        i = 2 * di + pr + 3
                        j = 2 * dj + pc + 3
                        if 0 <= i < 7 and 0 <= j < 7:
                            rows.append(conv1_w[:, c, i, j])
                        else:
                            rows.append(zero)
            taps.append(jnp.stack(rows))
    return jnp.stack(taps).astype(jnp.bfloat16)


def _stem_conv(x_nchw, conv1_w, bias, n, B):
    xs = x_nchw.astype(jnp.bfloat16).reshape(n, 3, 48, 2, 48, 2)
    xs = jnp.transpose(xs, (0, 2, 4, 1, 3, 5)).reshape(n, 48, 48, 12)
    xs = jnp.pad(xs, ((0, 0), (2, 1), (2, 6), (0, 0))).reshape(n * 2856, 12)
    w2 = _stem_w2(conv1_w)
    R = B * 2856
    grid = n // B
    out, st = pl.pallas_call(
        functools.partial(_stem_conv_kernel, B=B),
        out_shape=(jax.ShapeDtypeStruct((n * 2304, 64), jnp.float32),
                   jax.ShapeDtypeStruct((grid, 2, 64), jnp.float32)),
        grid_spec=pltpu.PrefetchScalarGridSpec(
            num_scalar_prefetch=0,
            grid=(grid,),
            in_specs=[pl.BlockSpec((R, 12), lambda i: (i, 0)),
                      pl.BlockSpec((16, 12, 64), lambda i: (0, 0, 0)),
                      pl.BlockSpec((1, 64), lambda i: (0, 0))],
            out_specs=(pl.BlockSpec((B * 2304, 64), lambda i: (i, 0)),
                       pl.BlockSpec((1, 2, 64), lambda i: (i, 0, 0))),
            scratch_shapes=[pltpu.VMEM((_ru(114, 8) + R + 57, 12),
                                       jnp.bfloat16)],
        ),
        compiler_params=pltpu.CompilerParams(
            dimension_semantics=("parallel",)),
    )(xs, w2, bias.astype(jnp.float32).reshape(1, 64))
    return out, jnp.sum(st, axis=0)


def _stem_pool_kernel(x_ref, s_ref, b_ref, o_ref, st_ref, *, B):
    C = x_ref.shape[-1]
    a = jnp.maximum(x_ref[...] * s_ref[...] + b_ref[...], 0.0)
    v = a.reshape(B * 24, 2, 48, C)
    pm = jnp.max(v, axis=1)
    od = v[:, 1].reshape(B, 24, 48, C)
    sh = jnp.concatenate([jnp.zeros_like(od[:, :1]), od[:, :-1]],
                         axis=1).reshape(B * 24, 48, C)
    rr = jnp.maximum(pm, sh)
    vw = rr.reshape(B * 24, 24, 2, C)
    pmw = jnp.max(vw, axis=2)
    odw = vw[:, :, 1, :]
    shw = jnp.concatenate([jnp.zeros_like(odw[:, :1]), odw[:, :-1]], axis=1)
    y16 = jnp.maximum(pmw, shw).reshape(B * 576, C).astype(jnp.bfloat16)
    o_ref[...] = y16
    st_ref[0] = _stats2(y16.astype(jnp.float32))


def _stem_pool(conv_out, scale, bias, n, B):
    C = conv_out.shape[-1]
    grid = n // B
    out, st = pl.pallas_call(
        functools.partial(_stem_pool_kernel, B=B),
        out_shape=(jax.ShapeDtypeStruct((n * 576, C), jnp.bfloat16),
                   jax.ShapeDtypeStruct((grid, 2, C), jnp.float32)),
        grid=(grid,),
        in_specs=[pl.BlockSpec((B * 2304, C), lambda i: (i, 0)),
                  pl.BlockSpec((1, C), lambda i: (0, 0)),
                  pl.BlockSpec((1, C), lambda i: (0, 0))],
        out_specs=(pl.BlockSpec((B * 576, C), lambda i: (i, 0)),
                   pl.BlockSpec((1, 2, C), lambda i: (i, 0, 0))),
        compiler_params=pltpu.CompilerParams(
            dimension_semantics=("parallel",)),
    )(conv_out, scale.astype(jnp.float32).reshape(1, C),
      bias.astype(jnp.float32).reshape(1, C))
    return out, jnp.sum(st, axis=0)


def _dense_kernel(x_ref, w_ref, s_ref, b_ref, m_ref, o_ref, st_ref, pad_ref,
                  *, width, g):
    rows = x_ref.shape[0]
    off = pad_ref.shape[0] - rows - (width + 1)

    @pl.when(pl.program_id(0) == 0)
    def _():
        pad_ref[...] = jnp.zeros_like(pad_ref)

    x = x_ref[...]
    a = jnp.maximum(x.astype(jnp.float32) * s_ref[...] + b_ref[...], 0.0)
    pad_ref[off:off + rows, :] = a.astype(jnp.bfloat16)

    acc = jnp.zeros((rows, g), jnp.float32)
    for di in range(3):
        for dj in range(3):
            tap = di * 3 + dj
            shift = off + (di - 1) * width + (dj - 1)
            src = pad_ref[shift:shift + rows, :]
            part = jnp.dot(src, w_ref[tap], preferred_element_type=jnp.float32)
            ok = m_ref[:, tap:tap + 1] > 0.5
            acc = acc + jnp.where(ok, part, 0.0)

    y16 = acc.astype(jnp.bfloat16)
    o_ref[...] = jnp.concatenate([y16, x], axis=-1)
    st_ref[0] = _stats2(y16.astype(jnp.float32))


def _tap_masks(h, w, B):
    r = jnp.repeat(jnp.arange(h), w)
    c = jnp.tile(jnp.arange(w), h)
    cols = []
    for di in (-1, 0, 1):
        for dj in (-1, 0, 1):
            cols.append((r + di >= 0) & (r + di < h) &
                        (c + dj >= 0) & (c + dj < w))
    m = jnp.stack(cols, axis=1).astype(jnp.float32)
    return jnp.tile(m, (B, 1))


def _dense_layer(slab, w_oihw, scale, bias, h, w, B, n):
    rows_total, C = slab.shape
    g = w_oihw.shape[0]
    m = h * w
    R = B * m
    off = _ru(w + 1, 8)
    pad_rows = off + R + w + 1
    grid = n // B
    wt = jnp.transpose(w_oihw, (2, 3, 1, 0)).reshape(9, C, g).astype(jnp.bfloat16)
    masks = _tap_masks(h, w, B)
    out, st = pl.pallas_call(
        functools.partial(_dense_kernel, width=w, g=g),
        out_shape=(jax.ShapeDtypeStruct((rows_total, C + g), jnp.bfloat16),
                   jax.ShapeDtypeStruct((grid, 2, g), jnp.float32)),
        grid_spec=pltpu.PrefetchScalarGridSpec(
            num_scalar_prefetch=0,
            grid=(grid,),
            in_specs=[pl.BlockSpec((R, C), lambda i: (i, 0)),
                      pl.BlockSpec((9, C, g), lambda i: (0, 0, 0)),
                      pl.BlockSpec((1, C), lambda i: (0, 0)),
                      pl.BlockSpec((1, C), lambda i: (0, 0)),
                      pl.BlockSpec((R, 9), lambda i: (0, 0))],
            out_specs=(pl.BlockSpec((R, C + g), lambda i: (i, 0)),
                       pl.BlockSpec((1, 2, g), lambda i: (i, 0, 0))),
            scratch_shapes=[pltpu.VMEM((pad_rows, C), jnp.bfloat16)],
        ),
        compiler_params=pltpu.CompilerParams(
            dimension_semantics=("parallel",)),
    )(slab, wt, scale.astype(jnp.float32).reshape(1, C),
      bias.astype(jnp.float32).reshape(1, C), masks)
    return out, jnp.sum(st, axis=0)


def _trans_kernel(x_ref, w_ref, s_ref, b_ref, ob_ref, o_ref, st_ref, *, B, h, w):
    a = jnp.maximum(x_ref[...].astype(jnp.float32) * s_ref[...] + b_ref[...],
                    0.0).astype(jnp.bfloat16)
    y = jnp.dot(a, w_ref[...], preferred_element_type=jnp.float32) + ob_ref[...]
    C = y.shape[-1]
    t = y.reshape(B * h // 2, 2, w, C).sum(axis=1)
    t = t.reshape(B * h // 2, w // 2, 2, C).sum(axis=2)
    y16 = (t.reshape(B * h * w // 4, C) * 0.25).astype(jnp.bfloat16)
    o_ref[...] = y16
    st_ref[0] = _stats2(y16.astype(jnp.float32))


def _transition(slab, w_oihw, cbias, scale, bias, h, w, B, n):
    rows_total, C = slab.shape
    Cout = w_oihw.shape[0]
    m = h * w
    grid = n // B
    wmat = w_oihw.reshape(Cout, C).T.astype(jnp.bfloat16)
    out, st = pl.pallas_call(
        functools.partial(_trans_kernel, B=B, h=h, w=w),
        out_shape=(jax.ShapeDtypeStruct((rows_total // 4, Cout), jnp.bfloat16),
                   jax.ShapeDtypeStruct((grid, 2, Cout), jnp.float32)),
        grid=(grid,),
        in_specs=[pl.BlockSpec((B * m, C), lambda i: (i, 0)),
                  pl.BlockSpec((C, Cout), lambda i: (0, 0)),
                  pl.BlockSpec((1, C), lambda i: (0, 0)),
                  pl.BlockSpec((1, C), lambda i: (0, 0)),
                  pl.BlockSpec((1, Cout), lambda i: (0, 0))],
        out_specs=(pl.BlockSpec((B * m // 4, Cout), lambda i: (i, 0)),
                   pl.BlockSpec((1, 2, Cout), lambda i: (i, 0, 0))),
        compiler_params=pltpu.CompilerParams(
            dimension_semantics=("parallel",)),
    )(slab, wmat, scale.astype(jnp.float32).reshape(1, C),
      bias.astype(jnp.float32).reshape(1, C),
      cbias.astype(jnp.float32).reshape(1, Cout))
    return out, jnp.sum(st, axis=0)


def _head_kernel(x_ref, s_ref, b_ref, w_ref, ob_ref, o_ref):
    rows, C = x_ref.shape
    a = jnp.maximum(x_ref[...].astype(jnp.float32) * s_ref[...] + b_ref[...], 0.0)
    f = a.reshape(rows // 9, 9, C).sum(axis=1) * (1.0 / 9.0)
    o_ref[...] = jnp.dot(f.astype(jnp.bfloat16), w_ref[...],
                         preferred_element_type=jnp.float32) + ob_ref[...]


def _head(slab, scale, bias, fc_w, fc_b, n):
    rows_total, C = slab.shape
    ncls = fc_w.shape[0]
    Np = _ru(ncls, 128)
    wmat = jnp.pad(fc_w.T.astype(jnp.bfloat16), ((0, 0), (0, Np - ncls)))
    ob = jnp.pad(fc_b.astype(jnp.float32), (0, Np - ncls)).reshape(1, Np)
    out = pl.pallas_call(
        _head_kernel,
        out_shape=jax.ShapeDtypeStruct((n, Np), jnp.float32),
        grid=(2,),
        in_specs=[pl.BlockSpec((rows_total // 2, C), lambda i: (i, 0)),
                  pl.BlockSpec((1, C), lambda i: (0, 0)),
                  pl.BlockSpec((1, C), lambda i: (0, 0)),
                  pl.BlockSpec((C, Np), lambda i: (0, 0)),
                  pl.BlockSpec((1, Np), lambda i: (0, 0))],
        out_specs=pl.BlockSpec((n // 2, Np), lambda i: (i, 0)),
        compiler_params=pltpu.CompilerParams(
            dimension_semantics=("parallel",)),
    )(slab, scale.astype(jnp.float32).reshape(1, C),
      bias.astype(jnp.float32).reshape(1, C), wmat, ob)
    return out[:, :ncls]


def kernel(x, conv1_w, conv1_b, bn1_g, bn1_b, b0_l0_bn_g, b0_l0_bn_b, b0_l0_w, b0_l1_bn_g, b0_l1_bn_b, b0_l1_w, b0_l2_bn_g, b0_l2_bn_b, b0_l2_w, b0_l3_bn_g, b0_l3_bn_b, b0_l3_w, t0_bn_g, t0_bn_b, t0_w, t0_b, b1_l0_bn_g, b1_l0_bn_b, b1_l0_w, b1_l1_bn_g, b1_l1_bn_b, b1_l1_w, b1_l2_bn_g, b1_l2_bn_b, b1_l2_w, b1_l3_bn_g, b1_l3_bn_b, b1_l3_w, t1_bn_g, t1_bn_b, t1_w, t1_b, b2_l0_bn_g, b2_l0_bn_b, b2_l0_w, b2_l1_bn_g, b2_l1_bn_b, b2_l1_w, b2_l2_bn_g, b2_l2_bn_b, b2_l2_w, b2_l3_bn_g, b2_l3_bn_b, b2_l3_w, t2_bn_g, t2_bn_b, t2_w, t2_b, b3_l0_bn_g, b3_l0_bn_b, b3_l0_w, b3_l1_bn_g, b3_l1_bn_b, b3_l1_w, b3_l2_bn_g, b3_l2_bn_b, b3_l2_w, b3_l3_bn_g, b3_l3_bn_b, b3_l3_w, bn2_g, bn2_b, fc_w, fc_b):
    n = x.shape[0]

    conv_out, st = _stem_conv(x, conv1_w, conv1_b, n, B=4)
    s1, b1 = _scale_bias(st[0], st[1], float(n * 48 * 48), bn1_g, bn1_b)
    slab, st0 = _stem_pool(conv_out, s1, b1, n, B=8)

    seg_sums, seg_sqs = [st0[0]], [st0[1]]
    count = float(n * 576)
    h = w = 24

    blocks = [
        [(b0_l0_bn_g, b0_l0_bn_b, b0_l0_w), (b0_l1_bn_g, b0_l1_bn_b, b0_l1_w),
         (b0_l2_bn_g, b0_l2_bn_b, b0_l2_w), (b0_l3_bn_g, b0_l3_bn_b, b0_l3_w)],
        [(b1_l0_bn_g, b1_l0_bn_b, b1_l0_w), (b1_l1_bn_g, b1_l1_bn_b, b1_l1_w),
         (b1_l2_bn_g, b1_l2_bn_b, b1_l2_w), (b1_l3_bn_g, b1_l3_bn_b, b1_l3_w)],
        [(b2_l0_bn_g, b2_l0_bn_b, b2_l0_w), (b2_l1_bn_g, b2_l1_bn_b, b2_l1_w),
         (b2_l2_bn_g, b2_l2_bn_b, b2_l2_w), (b2_l3_bn_g, b2_l3_bn_b, b2_l3_w)],
        [(b3_l0_bn_g, b3_l0_bn_b, b3_l0_w), (b3_l1_bn_g, b3_l1_bn_b, b3_l1_w),
         (b3_l2_bn_g, b3_l2_bn_b, b3_l2_w), (b3_l3_bn_g, b3_l3_bn_b, b3_l3_w)],
    ]
    transitions = [(t0_bn_g, t0_bn_b, t0_w, t0_b),
                   (t1_bn_g, t1_bn_b, t1_w, t1_b),
                   (t2_bn_g, t2_bn_b, t2_w, t2_b)]
    dense_B = [8, 16, 64, 128]
    trans_B = [8, 16, 64]

    for bi, layers in enumerate(blocks):
        for (g_, b_, w_) in layers:
            gsum = jnp.concatenate(seg_sums)
            gsq = jnp.concatenate(seg_sqs)
            sc, bs = _scale_bias(gsum, gsq, count, g_, b_)
            slab, st = _dense_layer(slab, w_, sc, bs, h, w, dense_B[bi], n)
            seg_sums.insert(0, st[0])
            seg_sqs.insert(0, st[1])
        if bi < 3:
            tg, tb, tw, tbb = transitions[bi]
            gsum = jnp.concatenate(seg_sums)
            gsq = jnp.concatenate(seg_sqs)
            sc, bs = _scale_bias(gsum, gsq, count, tg, tb)
            slab, st = _transition(slab, tw, tbb, sc, bs, h, w, trans_B[bi], n)
            h //= 2
            w //= 2
            count = float(n * h * w)
            seg_sums, seg_sqs = [st[0]], [st[1]]

    gsum = jnp.concatenate(seg_sums)
    gsq = jnp.concatenate(seg_sqs)
    sc, bs = _scale_bias(gsum, gsq, count, bn2_g, bn2_b)
    return _head(slab, sc, bs, fc_w, fc_b, n)


# speedup vs baseline: 3.7084x; 1.0223x over previous
import functools

import jax
import jax.numpy as jnp
from jax import lax
from jax.experimental import pallas as pl
from jax.experimental.pallas import tpu as pltpu


def _ru(x, m):
    return (x + m - 1) // m * m


def _scale_bias(csum, csumsq, count, gamma, beta, eps=1e-5):
    mean = csum / count
    var = csumsq / count - mean * mean
    s = gamma * lax.rsqrt(var + eps)
    return s, beta - mean * s


def _stats2(y):
    return jnp.concatenate([jnp.sum(y, axis=0, keepdims=True),
                            jnp.sum(y * y, axis=0, keepdims=True)], axis=0)


def _stem_conv_kernel(x_ref, w_ref, bias_ref, o_ref, st_ref, pad_ref, *, B):
    R = x_ref.shape[0]
    off = pad_ref.shape[0] - R - 57
    pad_ref[off:off + R, :] = x_ref[...]
    srcs = []
    for di in (-2, -1, 0, 1):
        for dj in (-2, -1, 0, 1):
            s = off + di * 56 + dj
            srcs.append(pad_ref[s:s + R, :])
    a = jnp.concatenate(srcs, axis=-1)
    y = jnp.dot(a, w_ref[...], preferred_element_type=jnp.float32) + bias_ref[...]
    yv = y.reshape(B, 51, 56, y.shape[-1])[:, 2:50, 2:50, :]
    yv = yv.reshape(B * 2304, y.shape[-1])
    o_ref[...] = yv
    st_ref[0] = _stats2(yv)


def _stem_w2(conv1_w):
    zero = jnp.zeros((64,), conv1_w.dtype)
    taps = []
    for di in (-2, -1, 0, 1):
        for dj in (-2, -1, 0, 1):
            rows = []
            for c in range(3):
                for pr in (0, 1):
                    for pc in (0, 1):
                        i = 2 * di + pr + 3
                        j = 2 * dj + pc + 3
                        if 0 <= i < 7 and 0 <= j < 7:
                            rows.append(conv1_w[:, c, i, j])
                        else:
                            rows.append(zero)
            taps.append(jnp.stack(rows))
    return jnp.stack(taps).astype(jnp.bfloat16)


def _stem_conv(x_nchw, conv1_w, bias, n, B):
    xs = x_nchw.astype(jnp.bfloat16).reshape(n, 3, 48, 2, 48, 2)
    xs = jnp.transpose(xs, (0, 2, 4, 1, 3, 5)).reshape(n, 48, 48, 12)
    xs = jnp.pad(xs, ((0, 0), (2, 1), (2, 6), (0, 0))).reshape(n * 2856, 12)
    w2 = _stem_w2(conv1_w).reshape(192, 64)
    R = B * 2856
    grid = n // B
    out, st = pl.pallas_call(
        functools.partial(_stem_conv_kernel, B=B),
        out_shape=(jax.ShapeDtypeStruct((n * 2304, 64), jnp.float32),
                   jax.ShapeDtypeStruct((grid, 2, 64), jnp.float32)),
        grid_spec=pltpu.PrefetchScalarGridSpec(
            num_scalar_prefetch=0,
            grid=(grid,),
            in_specs=[pl.BlockSpec((R, 12), lambda i: (i, 0)),
                      pl.BlockSpec((192, 64), lambda i: (0, 0)),
                      pl.BlockSpec((1, 64), lambda i: (0, 0))],
            out_specs=(pl.BlockSpec((B * 2304, 64), lambda i: (i, 0)),
                       pl.BlockSpec((1, 2, 64), lambda i: (i, 0, 0))),
            scratch_shapes=[pltpu.VMEM((_ru(114, 8) + R + 57, 12),
                                       jnp.bfloat16)],
        ),
        compiler_params=pltpu.CompilerParams(
            dimension_semantics=("parallel",)),
    )(xs, w2, bias.astype(jnp.float32).reshape(1, 64))
    return out, jnp.sum(st, axis=0)


def _stem_pool_kernel(x_ref, s_ref, b_ref, o_ref, st_ref, *, B):
    C = x_ref.shape[-1]
    a = jnp.maximum(x_ref[...] * s_ref[...] + b_ref[...], 0.0)
    v = a.reshape(B * 24, 2, 48, C)
    pm = jnp.max(v, axis=1)
    od = v[:, 1].reshape(B, 24, 48, C)
    sh = jnp.concatenate([jnp.zeros_like(od[:, :1]), od[:, :-1]],
                         axis=1).reshape(B * 24, 48, C)
    rr = jnp.maximum(pm, sh)
    vw = rr.reshape(B * 24, 24, 2, C)
    pmw = jnp.max(vw, axis=2)
    odw = vw[:, :, 1, :]
    shw = jnp.concatenate([jnp.zeros_like(odw[:, :1]), odw[:, :-1]], axis=1)
    y16 = jnp.maximum(pmw, shw).reshape(B * 576, C).astype(jnp.bfloat16)
    o_ref[...] = y16
    st_ref[0] = _stats2(y16.astype(jnp.float32))


def _stem_pool(conv_out, scale, bias, n, B):
    C = conv_out.shape[-1]
    grid = n // B
    out, st = pl.pallas_call(
        functools.partial(_stem_pool_kernel, B=B),
        out_shape=(jax.ShapeDtypeStruct((n * 576, C), jnp.bfloat16),
                   jax.ShapeDtypeStruct((grid, 2, C), jnp.float32)),
        grid=(grid,),
        in_specs=[pl.BlockSpec((B * 2304, C), lambda i: (i, 0)),
                  pl.BlockSpec((1, C), lambda i: (0, 0)),
                  pl.BlockSpec((1, C), lambda i: (0, 0))],
        out_specs=(pl.BlockSpec((B * 576, C), lambda i: (i, 0)),
                   pl.BlockSpec((1, 2, C), lambda i: (i, 0, 0))),
        compiler_params=pltpu.CompilerParams(
            dimension_semantics=("parallel",)),
    )(conv_out, scale.astype(jnp.float32).reshape(1, C),
      bias.astype(jnp.float32).reshape(1, C))
    return out, jnp.sum(st, axis=0)


def _dense_kernel(x_ref, w_ref, s_ref, b_ref, m_ref, o_ref, st_ref, pad_ref,
                  *, width, g):
    rows = x_ref.shape[0]
    off = pad_ref.shape[0] - rows - (width + 1)

    @pl.when(pl.program_id(0) == 0)
    def _():
        pad_ref[...] = jnp.zeros_like(pad_ref)

    x = x_ref[...]
    a = jnp.maximum(x.astype(jnp.float32) * s_ref[...] + b_ref[...], 0.0)
    pad_ref[off:off + rows, :] = a.astype(jnp.bfloat16)

    acc = jnp.zeros((rows, g), jnp.float32)
    for di in range(3):
        for dj in range(3):
            tap = di * 3 + dj
            shift = off + (di - 1) * width + (dj - 1)
            src = pad_ref[shift:shift + rows, :]
            part = jnp.dot(src, w_ref[tap], preferred_element_type=jnp.float32)
            ok = m_ref[:, tap:tap + 1] > 0.5
            acc = acc + jnp.where(ok, part, 0.0)

    y16 = acc.astype(jnp.bfloat16)
    o_ref[...] = jnp.concatenate([y16, x], axis=-1)
    st_ref[0] = _stats2(y16.astype(jnp.float32))


def _tap_masks(h, w, B):
    r = jnp.repeat(jnp.arange(h), w)
    c = jnp.tile(jnp.arange(w), h)
    cols = []
    for di in (-1, 0, 1):
        for dj in (-1, 0, 1):
            cols.append((r + di >= 0) & (r + di < h) &
                        (c + dj >= 0) & (c + dj < w))
    m = jnp.stack(cols, axis=1).astype(jnp.float32)
    return jnp.tile(m, (B, 1))


def _dense_layer(slab, w_oihw, scale, bias, h, w, B, n, masks):
    rows_total, C = slab.shape
    g = w_oihw.shape[0]
    m = h * w
    R = B * m
    off = _ru(w + 1, 8)
    pad_rows = off + R + w + 1
    grid = n // B
    wt = jnp.transpose(w_oihw, (2, 3, 1, 0)).reshape(9, C, g).astype(jnp.bfloat16)
    out, st = pl.pallas_call(
        functools.partial(_dense_kernel, width=w, g=g),
        out_shape=(jax.ShapeDtypeStruct((rows_total, C + g), jnp.bfloat16),
                   jax.ShapeDtypeStruct((grid, 2, g), jnp.float32)),
        grid_spec=pltpu.PrefetchScalarGridSpec(
            num_scalar_prefetch=0,
            grid=(grid,),
            in_specs=[pl.BlockSpec((R, C), lambda i: (i, 0)),
                      pl.BlockSpec((9, C, g), lambda i: (0, 0, 0)),
                      pl.BlockSpec((1, C), lambda i: (0, 0)),
                      pl.BlockSpec((1, C), lambda i: (0, 0)),
                      pl.BlockSpec((R, 9), lambda i: (0, 0))],
            out_specs=(pl.BlockSpec((R, C + g), lambda i: (i, 0)),
                       pl.BlockSpec((1, 2, g), lambda i: (i, 0, 0))),
            scratch_shapes=[pltpu.VMEM((pad_rows, C), jnp.bfloat16)],
        ),
        compiler_params=pltpu.CompilerParams(
            dimension_semantics=("parallel",)),
    )(slab, wt, scale.astype(jnp.float32).reshape(1, C),
      bias.astype(jnp.float32).reshape(1, C), masks)
    return out, jnp.sum(st, axis=0)


def _trans_kernel(x_ref, w_ref, s_ref, b_ref, ob_ref, o_ref, st_ref, *, B, h, w):
    a = jnp.maximum(x_ref[...].astype(jnp.float32) * s_ref[...] + b_ref[...],
                    0.0).astype(jnp.bfloat16)
    y = jnp.dot(a, w_ref[...], preferred_element_type=jnp.float32) + ob_ref[...]
    C = y.shape[-1]
    t = y.reshape(B * h // 2, 2, w, C).sum(axis=1)
    t = t.reshape(B * h // 2, w // 2, 2, C).sum(axis=2)
    y16 = (t.reshape(B * h * w // 4, C) * 0.25).astype(jnp.bfloat16)
    o_ref[...] = y16
    st_ref[0] = _stats2(y16.astype(jnp.float32))


def _transition(slab, w_oihw, cbias, scale, bias, h, w, B, n):
    rows_total, C = slab.shape
    Cout = w_oihw.shape[0]
    m = h * w
    grid = n // B
    wmat = w_oihw.reshape(Cout, C).T.astype(jnp.bfloat16)
    out, st = pl.pallas_call(
        functools.partial(_trans_kernel, B=B, h=h, w=w),
        out_shape=(jax.ShapeDtypeStruct((rows_total // 4, Cout), jnp.bfloat16),
                   jax.ShapeDtypeStruct((grid, 2, Cout), jnp.float32)),
        grid=(grid,),
        in_specs=[pl.BlockSpec((B * m, C), lambda i: (i, 0)),
                  pl.BlockSpec((C, Cout), lambda i: (0, 0)),
                  pl.BlockSpec((1, C), lambda i: (0, 0)),
                  pl.BlockSpec((1, C), lambda i: (0, 0)),
                  pl.BlockSpec((1, Cout), lambda i: (0, 0))],
        out_specs=(pl.BlockSpec((B * m // 4, Cout), lambda i: (i, 0)),
                   pl.BlockSpec((1, 2, Cout), lambda i: (i, 0, 0))),
        compiler_params=pltpu.CompilerParams(
            dimension_semantics=("parallel",)),
    )(slab, wmat, scale.astype(jnp.float32).reshape(1, C),
      bias.astype(jnp.float32).reshape(1, C),
      cbias.astype(jnp.float32).reshape(1, Cout))
    return out, jnp.sum(st, axis=0)


def _head_kernel(x_ref, s_ref, b_ref, w_ref, ob_ref, o_ref):
    rows, C = x_ref.shape
    a = jnp.maximum(x_ref[...].astype(jnp.float32) * s_ref[...] + b_ref[...], 0.0)
    f = a.reshape(rows // 9, 9, C).sum(axis=1) * (1.0 / 9.0)
    o_ref[...] = jnp.dot(f.astype(jnp.bfloat16), w_ref[...],
                         preferred_element_type=jnp.float32) + ob_ref[...]


def _head(slab, scale, bias, fc_w, fc_b, n):
    rows_total, C = slab.shape
    ncls = fc_w.shape[0]
    Np = _ru(ncls, 128)
    wmat = jnp.pad(fc_w.T.astype(jnp.bfloat16), ((0, 0), (0, Np - ncls)))
    ob = jnp.pad(fc_b.astype(jnp.float32), (0, Np - ncls)).reshape(1, Np)
    out = pl.pallas_call(
        _head_kernel,
        out_shape=jax.ShapeDtypeStruct((n, Np), jnp.float32),
        grid=(2,),
        in_specs=[pl.BlockSpec((rows_total // 2, C), lambda i: (i, 0)),
                  pl.BlockSpec((1, C), lambda i: (0, 0)),
                  pl.BlockSpec((1, C), lambda i: (0, 0)),
                  pl.BlockSpec((C, Np), lambda i: (0, 0)),
                  pl.BlockSpec((1, Np), lambda i: (0, 0))],
        out_specs=pl.BlockSpec((n // 2, Np), lambda i: (i, 0)),
        compiler_params=pltpu.CompilerParams(
            dimension_semantics=("parallel",)),
    )(slab, scale.astype(jnp.float32).reshape(1, C),
      bias.astype(jnp.float32).reshape(1, C), wmat, ob)
    return out[:, :ncls]


def kernel(x, conv1_w, conv1_b, bn1_g, bn1_b, b0_l0_bn_g, b0_l0_bn_b, b0_l0_w, b0_l1_bn_g, b0_l1_bn_b, b0_l1_w, b0_l2_bn_g, b0_l2_bn_b, b0_l2_w, b0_l3_bn_g, b0_l3_bn_b, b0_l3_w, t0_bn_g, t0_bn_b, t0_w, t0_b, b1_l0_bn_g, b1_l0_bn_b, b1_l0_w, b1_l1_bn_g, b1_l1_bn_b, b1_l1_w, b1_l2_bn_g, b1_l2_bn_b, b1_l2_w, b1_l3_bn_g, b1_l3_bn_b, b1_l3_w, t1_bn_g, t1_bn_b, t1_w, t1_b, b2_l0_bn_g, b2_l0_bn_b, b2_l0_w, b2_l1_bn_g, b2_l1_bn_b, b2_l1_w, b2_l2_bn_g, b2_l2_bn_b, b2_l2_w, b2_l3_bn_g, b2_l3_bn_b, b2_l3_w, t2_bn_g, t2_bn_b, t2_w, t2_b, b3_l0_bn_g, b3_l0_bn_b, b3_l0_w, b3_l1_bn_g, b3_l1_bn_b, b3_l1_w, b3_l2_bn_g, b3_l2_bn_b, b3_l2_w, b3_l3_bn_g, b3_l3_bn_b, b3_l3_w, bn2_g, bn2_b, fc_w, fc_b):
    n = x.shape[0]

    conv_out, st = _stem_conv(x, conv1_w, conv1_b, n, B=2)
    s1, b1 = _scale_bias(st[0], st[1], float(n * 48 * 48), bn1_g, bn1_b)
    slab, st0 = _stem_pool(conv_out, s1, b1, n, B=8)

    seg_sums, seg_sqs = [st0[0]], [st0[1]]
    count = float(n * 576)
    h = w = 24

    blocks = [
        [(b0_l0_bn_g, b0_l0_bn_b, b0_l0_w), (b0_l1_bn_g, b0_l1_bn_b, b0_l1_w),
         (b0_l2_bn_g, b0_l2_bn_b, b0_l2_w), (b0_l3_bn_g, b0_l3_bn_b, b0_l3_w)],
        [(b1_l0_bn_g, b1_l0_bn_b, b1_l0_w), (b1_l1_bn_g, b1_l1_bn_b, b1_l1_w),
         (b1_l2_bn_g, b1_l2_bn_b, b1_l2_w), (b1_l3_bn_g, b1_l3_bn_b, b1_l3_w)],
        [(b2_l0_bn_g, b2_l0_bn_b, b2_l0_w), (b2_l1_bn_g, b2_l1_bn_b, b2_l1_w),
         (b2_l2_bn_g, b2_l2_bn_b, b2_l2_w), (b2_l3_bn_g, b2_l3_bn_b, b2_l3_w)],
        [(b3_l0_bn_g, b3_l0_bn_b, b3_l0_w), (b3_l1_bn_g, b3_l1_bn_b, b3_l1_w),
         (b3_l2_bn_g, b3_l2_bn_b, b3_l2_w), (b3_l3_bn_g, b3_l3_bn_b, b3_l3_w)],
    ]
    transitions = [(t0_bn_g, t0_bn_b, t0_w, t0_b),
                   (t1_bn_g, t1_bn_b, t1_w, t1_b),
                   (t2_bn_g, t2_bn_b, t2_w, t2_b)]
    dense_B = [8, 16, 64, 128]
    trans_B = [8, 16, 64]

    for bi, layers in enumerate(blocks):
        masks = _tap_masks(h, w, dense_B[bi])
        for (g_, b_, w_) in layers:
            gsum = jnp.concatenate(seg_sums)
            gsq = jnp.concatenate(seg_sqs)
            sc, bs = _scale_bias(gsum, gsq, count, g_, b_)
            slab, st = _dense_layer(slab, w_, sc, bs, h, w, dense_B[bi], n, masks)
            seg_sums.insert(0, st[0])
            seg_sqs.insert(0, st[1])
        if bi < 3:
            tg, tb, tw, tbb = transitions[bi]
            gsum = jnp.concatenate(seg_sums)
            gsq = jnp.concatenate(seg_sqs)
            sc, bs = _scale_bias(gsum, gsq, count, tg, tb)
            slab, st = _transition(slab, tw, tbb, sc, bs, h, w, trans_B[bi], n)
            h //= 2
            w //= 2
            count = float(n * h * w)
            seg_sums, seg_sqs = [st[0]], [st[1]]

    gsum = jnp.concatenate(seg_sums)
    gsq = jnp.concatenate(seg_sqs)
    sc, bs = _scale_bias(gsum, gsq, count, bn2_g, bn2_b)
    return _head(slab, sc, bs, fc_w, fc_b, n)
